# baseline (speedup 1.0000x reference)
import functools

import jax
import jax.numpy as jnp
import numpy as np
from jax import lax
from jax.experimental import pallas as pl
from jax.experimental.pallas import tpu as pltpu
from jax.experimental.pallas import tpu_sc as plsc

N_NODES = 10000
N_EDGES = 320000
D_IN = 128
D_EDGE = 4
D_OUT = 128
D_RADIAL = 10
H_RADIAL = 64
_SILU_NORM = 1.679177

NC = 2
NS = 16
L = 16
NW = NC * NS
EPW = N_EDGES // NW
EB = 40
N_ITERS = EPW // EB
N_PAD = 10112
ROWS_PER_TILE = N_PAD // NS
ZROWS = 158


def _y_body(x_ref, w_ref, y_ref):
    y_ref[...] = jnp.dot(x_ref[...], w_ref[...],
                         preferred_element_type=jnp.float32
                         ).astype(jnp.bfloat16)


def _coeff_body(emb_ref, attr_ref, w1_ref, w2_ref, out_ref):
    h = jnp.dot(emb_ref[...], w1_ref[...],
                preferred_element_type=jnp.float32) * (1.0 / np.sqrt(D_RADIAL))
    h = jax.nn.silu(h) * _SILU_NORM
    scale = 1.0 / (np.sqrt(H_RADIAL) * np.sqrt(D_IN * D_EDGE) * np.sqrt(32.0))
    r = jnp.sum(h * w2_ref[...], axis=1, keepdims=True) * scale
    out_ref[...] = attr_ref[...] * r


def _combine_body(p_ref, o_ref):
    o_ref[...] = p_ref[0] + p_ref[1]


MW = EB * 2


def _sc_body(y_hbm, meta_hbm, coeff_hbm, out_hbm,
             meta0, meta1, cf0, cf1, rows0, rows1, msg_v, acc_sh,
             msem0, msem1, rsem0, rsem1):
    cid = lax.axis_index("c")
    sid = lax.axis_index("s")
    wid = cid * NS + sid
    bid0 = wid * N_ITERS

    metas = (meta0, meta1)
    cfs = (cf0, cf1)
    rows = (rows0, rows1)
    msems = (msem0, msem1)
    rsems = (rsem0, rsem1)

    def _fire_meta(g, b):
        pltpu.async_copy(meta_hbm.at[pl.ds((bid0 + g) * MW, MW)],
                         metas[b], msems[b])
        pltpu.async_copy(
            coeff_hbm.at[pl.ds((bid0 + g) * EB * D_EDGE, EB * D_EDGE)],
            cfs[b], msems[b])

    def _wait_meta(g, b):
        pltpu.make_async_copy(meta_hbm.at[pl.ds((bid0 + g) * MW, MW)],
                              metas[b], msems[b]).wait()
        pltpu.make_async_copy(
            coeff_hbm.at[pl.ds((bid0 + g) * EB * D_EDGE, EB * D_EDGE)],
            cfs[b], msems[b]).wait()

    def _fire_rows(g, b):
        pltpu.async_copy(y_hbm.at[metas[b].at[pl.ds(0, EB)]],
                         rows[b], rsems[b])

    def _wait_rows(g, b):
        pltpu.make_async_copy(y_hbm.at[metas[b].at[pl.ds(0, EB)]],
                              rows[b], rsems[b]).wait()

    _fire_meta(0, 0)
    _fire_meta(1, 1)

    def _zero_row(r, carry):
        for k in range(D_OUT // L):
            msg_v[r, pl.ds(k * L, L)] = jnp.zeros((L,), jnp.float32)
        return carry
    lax.fori_loop(0, EB, _zero_row, 0)
    row0 = sid * ROWS_PER_TILE
    for z in range(ROWS_PER_TILE // EB):
        pltpu.sync_copy(msg_v, acc_sh.at[pl.ds(row0 + z * EB, EB), :])
    pltpu.sync_copy(msg_v,
                    acc_sh.at[pl.ds(row0 + ROWS_PER_TILE - EB, EB), :])
    plsc.subcore_barrier()

    _wait_meta(0, 0)
    _fire_rows(0, 0)

    def _block2(i2, carry):
        for b in range(2):
            g = i2 * 2 + b
            _wait_rows(g, b)

            @pl.when(g + 1 < N_ITERS)
            def _():
                _wait_meta(g + 1, 1 - b)
                _fire_rows(g + 1, 1 - b)

            def _edge4(e4, c):
                cvec = cfs[b][pl.ds(e4 * 4 * D_EDGE, L)]
                for u in range(4):
                    e = e4 * 4 + u
                    acc = [None] * (D_OUT // L)
                    for j in range(D_EDGE):
                        cj = jnp.full((L,), cvec[u * D_EDGE + j], jnp.float32)
                        for t in range(D_OUT // (2 * L)):
                            v32 = rows[b][e, pl.ds(j * 64 + t * L, L)]
                            v = plsc.bitcast(v32, jnp.bfloat16)
                            ya, yb = plsc.unpack(
                                v, format=plsc.PackFormat.INTERLEAVED,
                                preferred_element_type=jnp.float32)
                            if j == 0:
                                acc[2 * t] = cj * ya
                                acc[2 * t + 1] = cj * yb
                            else:
                                acc[2 * t] = acc[2 * t] + cj * ya
                                acc[2 * t + 1] = acc[2 * t + 1] + cj * yb
                    for k in range(D_OUT // L):
                        msg_v[e, pl.ds(k * L, L)] = acc[k]
                return c
            lax.fori_loop(0, EB // 4, _edge4, 0)

            pltpu.sync_copy(msg_v, acc_sh.at[metas[b].at[pl.ds(EB, EB)]],
                            add=True)

            @pl.when(g + 2 < N_ITERS)
            def _():
                _fire_meta(g + 2, b)
        return carry
    lax.fori_loop(0, N_ITERS // 2, _block2, 0)

    plsc.subcore_barrier()
    pltpu.sync_copy(acc_sh.at[pl.ds(row0, ROWS_PER_TILE), :],
                    out_hbm.at[cid, pl.ds(row0, ROWS_PER_TILE), :])


def kernel(node_in, edge_src, edge_dst, edge_attr, edge_len_emb,
           tp_weight, fc_w1, fc_w2):
    w2d = tp_weight.reshape(D_IN, D_EDGE * D_OUT)

    y = pl.pallas_call(
        _y_body,
        grid=(10,),
        in_specs=[
            pl.BlockSpec((N_NODES // 10, D_IN), lambda i: (i, 0)),
            pl.BlockSpec((D_IN, D_EDGE * D_OUT), lambda i: (0, 0)),
        ],
        out_specs=pl.BlockSpec((N_NODES // 10, D_EDGE * D_OUT),
                               lambda i: (i, 0)),
        out_shape=jax.ShapeDtypeStruct((N_NODES, D_EDGE * D_OUT),
                                       jnp.bfloat16),
    )(node_in, w2d)

    eb = N_EDGES // 40
    coeff = pl.pallas_call(
        _coeff_body,
        grid=(40,),
        in_specs=[
            pl.BlockSpec((eb, D_RADIAL), lambda i: (i, 0)),
            pl.BlockSpec((eb, D_EDGE), lambda i: (i, 0)),
            pl.BlockSpec((D_RADIAL, H_RADIAL), lambda i: (0, 0)),
            pl.BlockSpec((1, H_RADIAL), lambda i: (0, 0)),
        ],
        out_specs=pl.BlockSpec((eb, D_EDGE), lambda i: (i, 0)),
        out_shape=jax.ShapeDtypeStruct((N_EDGES, D_EDGE), jnp.float32),
    )(edge_len_emb, edge_attr, fc_w1, fc_w2.reshape(1, H_RADIAL))

    src_b = edge_src.reshape(-1, EB)
    dst_b = edge_dst.reshape(-1, EB)
    meta = jnp.concatenate([src_b, dst_b], axis=1).reshape(-1)
    coeff_flat = coeff.reshape(N_EDGES * D_EDGE)

    mesh = plsc.VectorSubcoreMesh(core_axis_name="c", subcore_axis_name="s",
                                  num_cores=NC, num_subcores=NS)
    y3 = lax.bitcast_convert_type(
        y.reshape(N_NODES, D_EDGE * D_OUT // 2, 2), jnp.int32)

    partials = pl.kernel(
        _sc_body,
        out_type=jax.ShapeDtypeStruct((NC, N_PAD, D_OUT), jnp.float32),
        mesh=mesh,
        compiler_params=pltpu.CompilerParams(needs_layout_passes=False),
        scratch_types=[
            pltpu.VMEM((MW,), jnp.int32),
            pltpu.VMEM((MW,), jnp.int32),
            pltpu.VMEM((EB * D_EDGE,), jnp.float32),
            pltpu.VMEM((EB * D_EDGE,), jnp.float32),
            pltpu.VMEM((EB, D_EDGE * D_OUT // 2), jnp.int32),
            pltpu.VMEM((EB, D_EDGE * D_OUT // 2), jnp.int32),
            pltpu.VMEM((EB, D_OUT), jnp.float32),
            pltpu.VMEM_SHARED((N_PAD, D_OUT), jnp.float32),
            pltpu.SemaphoreType.DMA,
            pltpu.SemaphoreType.DMA,
            pltpu.SemaphoreType.DMA,
            pltpu.SemaphoreType.DMA,
        ],
    )(y3, meta, coeff_flat)

    out = pl.pallas_call(
        _combine_body,
        grid=(10,),
        in_specs=[pl.BlockSpec((NC, N_NODES // 10, D_OUT),
                               lambda i: (0, i, 0))],
        out_specs=pl.BlockSpec((N_NODES // 10, D_OUT), lambda i: (i, 0)),
        out_shape=jax.ShapeDtypeStruct((N_NODES, D_OUT), jnp.float32),
    )(partials)
    out = out.reshape(N_NODES, 4, 2, L).transpose(0, 1, 3, 2)
    return out.reshape(N_NODES, D_OUT)


# speedup vs baseline: 1.2468x; 1.2468x over previous
import jax
import jax.numpy as jnp
import numpy as np
from jax import lax
from jax.experimental import pallas as pl
from jax.experimental.pallas import tpu as pltpu
from jax.experimental.pallas import tpu_sc as plsc

N_NODES = 10000
N_EDGES = 320000
D_IN = 128
D_EDGE = 4
D_OUT = 128
D_RADIAL = 10
H_RADIAL = 64
_SILU_NORM = 1.679177

NC = 2
NS = 16
L = 16
NW = NC * NS
EPW = N_EDGES // NW
EB = 40
N_ITERS = EPW // EB
N_PAD = 10112
ROWS_PER_TILE = N_PAD // NS
MW = EB * 2


def _y_body(x_ref, w_ref, y_ref):
    y_ref[...] = jnp.dot(x_ref[...], w_ref[...],
                         preferred_element_type=jnp.float32)


def _coeff_body(emb_ref, attr_ref, w1_ref, w2_ref, out_ref):
    h = jnp.dot(emb_ref[...], w1_ref[...],
                preferred_element_type=jnp.float32) * (1.0 / np.sqrt(D_RADIAL))
    h = jax.nn.silu(h) * _SILU_NORM
    scale = 1.0 / (np.sqrt(H_RADIAL) * np.sqrt(D_IN * D_EDGE) * np.sqrt(32.0))
    r = jnp.sum(h * w2_ref[...], axis=1, keepdims=True) * scale
    out_ref[...] = attr_ref[...] * r


def _combine_body(p_ref, o_ref):
    o_ref[...] = p_ref[0] + p_ref[1]


def _sc_body(y_hbm, meta_hbm, coeff_hbm, out_hbm,
             meta0, meta1, cf0, cf1, rows0, rows1, msg_v, acc_sh,
             msem0, msem1, rsem0, rsem1):
    cid = lax.axis_index("c")
    sid = lax.axis_index("s")
    wid = cid * NS + sid
    bid0 = wid * N_ITERS

    metas = (meta0, meta1)
    cfs = (cf0, cf1)
    rows = (rows0, rows1)
    msems = (msem0, msem1)
    rsems = (rsem0, rsem1)

    def _fire_meta(g, b):
        pltpu.async_copy(meta_hbm.at[pl.ds((bid0 + g) * MW, MW)],
                         metas[b], msems[b])
        pltpu.async_copy(
            coeff_hbm.at[pl.ds((bid0 + g) * EB * D_EDGE, EB * D_EDGE)],
            cfs[b], msems[b])

    def _wait_meta(g, b):
        pltpu.make_async_copy(meta_hbm.at[pl.ds((bid0 + g) * MW, MW)],
                              metas[b], msems[b]).wait()
        pltpu.make_async_copy(
            coeff_hbm.at[pl.ds((bid0 + g) * EB * D_EDGE, EB * D_EDGE)],
            cfs[b], msems[b]).wait()

    def _fire_rows(g, b):
        pltpu.async_copy(y_hbm.at[metas[b].at[pl.ds(0, EB)]],
                         rows[b], rsems[b])

    def _wait_rows(g, b):
        pltpu.make_async_copy(y_hbm.at[metas[b].at[pl.ds(0, EB)]],
                              rows[b], rsems[b]).wait()

    _fire_meta(0, 0)
    _fire_meta(1, 1)

    def _zero_row(r, carry):
        for k in range(D_OUT // L):
            msg_v[r, pl.ds(k * L, L)] = jnp.zeros((L,), jnp.float32)
        return carry
    lax.fori_loop(0, EB, _zero_row, 0)
    row0 = sid * ROWS_PER_TILE
    for z in range(ROWS_PER_TILE // EB):
        pltpu.sync_copy(msg_v, acc_sh.at[pl.ds(row0 + z * EB, EB), :])
    pltpu.sync_copy(msg_v,
                    acc_sh.at[pl.ds(row0 + ROWS_PER_TILE - EB, EB), :])
    plsc.subcore_barrier()

    _wait_meta(0, 0)
    _fire_rows(0, 0)

    def _block2(i2, carry):
        for b in range(2):
            g = i2 * 2 + b
            _wait_rows(g, b)

            @pl.when(g + 1 < N_ITERS)
            def _():
                _wait_meta(g + 1, 1 - b)
                _fire_rows(g + 1, 1 - b)

            @plsc.parallel_loop(0, EB // 4, unroll=2)
            def _edge4(e4):
                cvec = cfs[b][pl.ds(e4 * 4 * D_EDGE, L)]
                for u in range(4):
                    e = e4 * 4 + u
                    acc = [None] * (D_OUT // L)
                    for j in range(D_EDGE):
                        cj = jnp.full((L,), cvec[u * D_EDGE + j], jnp.float32)
                        for k in range(D_OUT // L):
                            y = rows[b][e, pl.ds(j * D_OUT + k * L, L)]
                            acc[k] = cj * y if j == 0 else acc[k] + cj * y
                    for k in range(D_OUT // L):
                        msg_v[e, pl.ds(k * L, L)] = acc[k]

            pltpu.sync_copy(msg_v, acc_sh.at[metas[b].at[pl.ds(EB, EB)]],
                            add=True)

            @pl.when(g + 2 < N_ITERS)
            def _():
                _fire_meta(g + 2, b)
        return carry
    lax.fori_loop(0, N_ITERS // 2, _block2, 0)

    plsc.subcore_barrier()
    pltpu.sync_copy(acc_sh.at[pl.ds(row0, ROWS_PER_TILE), :],
                    out_hbm.at[cid, pl.ds(row0, ROWS_PER_TILE), :])


def kernel(node_in, edge_src, edge_dst, edge_attr, edge_len_emb,
           tp_weight, fc_w1, fc_w2):
    w2d = tp_weight.reshape(D_IN, D_EDGE * D_OUT)

    y = pl.pallas_call(
        _y_body,
        grid=(10,),
        in_specs=[
            pl.BlockSpec((N_NODES // 10, D_IN), lambda i: (i, 0)),
            pl.BlockSpec((D_IN, D_EDGE * D_OUT), lambda i: (0, 0)),
        ],
        out_specs=pl.BlockSpec((N_NODES // 10, D_EDGE * D_OUT),
                               lambda i: (i, 0)),
        out_shape=jax.ShapeDtypeStruct((N_NODES, D_EDGE * D_OUT),
                                       jnp.float32),
    )(node_in, w2d)

    eb = N_EDGES // 40
    coeff = pl.pallas_call(
        _coeff_body,
        grid=(40,),
        in_specs=[
            pl.BlockSpec((eb, D_RADIAL), lambda i: (i, 0)),
            pl.BlockSpec((eb, D_EDGE), lambda i: (i, 0)),
            pl.BlockSpec((D_RADIAL, H_RADIAL), lambda i: (0, 0)),
            pl.BlockSpec((1, H_RADIAL), lambda i: (0, 0)),
        ],
        out_specs=pl.BlockSpec((eb, D_EDGE), lambda i: (i, 0)),
        out_shape=jax.ShapeDtypeStruct((N_EDGES, D_EDGE), jnp.float32),
    )(edge_len_emb, edge_attr, fc_w1, fc_w2.reshape(1, H_RADIAL))

    src_b = edge_src.reshape(-1, EB)
    dst_b = edge_dst.reshape(-1, EB)
    meta = jnp.concatenate([src_b, dst_b], axis=1).reshape(-1)
    coeff_flat = coeff.reshape(N_EDGES * D_EDGE)

    mesh = plsc.VectorSubcoreMesh(core_axis_name="c", subcore_axis_name="s",
                                  num_cores=NC, num_subcores=NS)
    partials = pl.kernel(
        _sc_body,
        out_type=jax.ShapeDtypeStruct((NC, N_PAD, D_OUT), jnp.float32),
        mesh=mesh,
        compiler_params=pltpu.CompilerParams(needs_layout_passes=False),
        scratch_types=[
            pltpu.VMEM((MW,), jnp.int32),
            pltpu.VMEM((MW,), jnp.int32),
            pltpu.VMEM((EB * D_EDGE,), jnp.float32),
            pltpu.VMEM((EB * D_EDGE,), jnp.float32),
            pltpu.VMEM((EB, D_EDGE * D_OUT), jnp.float32),
            pltpu.VMEM((EB, D_EDGE * D_OUT), jnp.float32),
            pltpu.VMEM((EB, D_OUT), jnp.float32),
            pltpu.VMEM_SHARED((N_PAD, D_OUT), jnp.float32),
            pltpu.SemaphoreType.DMA,
            pltpu.SemaphoreType.DMA,
            pltpu.SemaphoreType.DMA,
            pltpu.SemaphoreType.DMA,
        ],
    )(y, meta, coeff_flat)

    out = pl.pallas_call(
        _combine_body,
        grid=(10,),
        in_specs=[pl.BlockSpec((NC, N_NODES // 10, D_OUT),
                               lambda i: (0, i, 0))],
        out_specs=pl.BlockSpec((N_NODES // 10, D_OUT), lambda i: (i, 0)),
        out_shape=jax.ShapeDtypeStruct((N_NODES, D_OUT), jnp.float32),
    )(partials)
    return out
